# Initial kernel scaffold; baseline (speedup 1.0000x reference)
import jax
import jax.numpy as jnp
from jax.experimental import pallas as pl


def kernel(x, Wr, br, W1, b1, W2, b2, W3, b3):
    raise NotImplementedError("write your pallas kernel here")


# speedup vs baseline: 1.2849x; 1.2849x over previous
import jax
import jax.numpy as jnp
from jax.experimental import pallas as pl
from jax.experimental.pallas import tpu as pltpu

_N, _D, _E, _K, _C, _H1, _H2 = 2048, 1024, 8, 2, 50, 512, 256


def _router_body(x_ref, wr_ref, br_ref, probs_ref, wfull_ref):
    x = x_ref[...]
    scores = jnp.dot(x, wr_ref[...], preferred_element_type=jnp.float32)
    scores = scores + br_ref[...]
    m = jnp.max(scores, axis=1, keepdims=True)
    ex = jnp.exp(scores - m)
    s = jnp.sum(ex, axis=1, keepdims=True)
    probs = ex / s
    iota = jax.lax.broadcasted_iota(jnp.int32, probs.shape, 1)
    m1 = jnp.max(probs, axis=1, keepdims=True)
    i1 = jnp.min(jnp.where(probs == m1, iota, _E), axis=1, keepdims=True)
    pm = jnp.where(iota == i1, -1.0, probs)
    m2 = jnp.max(pm, axis=1, keepdims=True)
    i2 = jnp.min(jnp.where(pm == m2, iota, _E), axis=1, keepdims=True)
    wfull = jnp.where(iota == i1, m1, 0.0) + jnp.where(iota == i2, m2, 0.0)
    probs_ref[...] = probs
    wfull_ref[...] = wfull * (1.0 / _K)


def _expert_body(wfull_ref, x_ref, w1_ref, b1_ref, w2_ref, b2_ref, w3_ref,
                 b3_ref, out_ref):
    e = pl.program_id(0)
    x = x_ref[...]
    h1 = jnp.maximum(
        jnp.dot(x, w1_ref[0], preferred_element_type=jnp.float32) + b1_ref[0], 0.0)
    h2 = jnp.maximum(
        jnp.dot(h1, w2_ref[0], preferred_element_type=jnp.float32) + b2_ref[0], 0.0)
    o = jnp.dot(h2, w3_ref[0], preferred_element_type=jnp.float32) + b3_ref[0]
    onehot = (jax.lax.broadcasted_iota(jnp.int32, (1, _E), 1) == e).astype(jnp.float32)
    w_e = jnp.sum(wfull_ref[...] * onehot, axis=1, keepdims=True)
    acc = w_e * o

    @pl.when(e == 0)
    def _init():
        out_ref[...] = acc

    @pl.when(e > 0)
    def _accum():
        out_ref[...] = out_ref[...] + acc


def kernel(x, Wr, br, W1, b1, W2, b2, W3, b3):
    n, d = x.shape
    probs, wfull = pl.pallas_call(
        _router_body,
        grid=(1,),
        in_specs=[
            pl.BlockSpec((n, d), lambda i: (0, 0)),
            pl.BlockSpec((d, _E), lambda i: (0, 0)),
            pl.BlockSpec((1, _E), lambda i: (0, 0)),
        ],
        out_specs=[
            pl.BlockSpec((n, _E), lambda i: (0, 0)),
            pl.BlockSpec((n, _E), lambda i: (0, 0)),
        ],
        out_shape=[
            jax.ShapeDtypeStruct((n, _E), jnp.float32),
            jax.ShapeDtypeStruct((n, _E), jnp.float32),
        ],
    )(x, Wr, br.reshape(1, _E))

    out = pl.pallas_call(
        _expert_body,
        grid=(_E,),
        in_specs=[
            pl.BlockSpec((n, _E), lambda e: (0, 0)),
            pl.BlockSpec((n, d), lambda e: (0, 0)),
            pl.BlockSpec((1, _D, _H1), lambda e: (e, 0, 0)),
            pl.BlockSpec((1, 1, _H1), lambda e: (e, 0, 0)),
            pl.BlockSpec((1, _H1, _H2), lambda e: (e, 0, 0)),
            pl.BlockSpec((1, 1, _H2), lambda e: (e, 0, 0)),
            pl.BlockSpec((1, _H2, _C), lambda e: (e, 0, 0)),
            pl.BlockSpec((1, 1, _C), lambda e: (e, 0, 0)),
        ],
        out_specs=pl.BlockSpec((n, _C), lambda e: (0, 0)),
        out_shape=jax.ShapeDtypeStruct((n, _C), jnp.float32),
        compiler_params=pltpu.CompilerParams(
            dimension_semantics=("arbitrary",),
        ),
    )(wfull, x, W1, b1.reshape(_E, 1, _H1), W2, b2.reshape(_E, 1, _H2), W3,
      b3.reshape(_E, 1, _C))
    return (out, probs)
